# baseline (speedup 1.0000x reference)
import functools

import jax
import jax.numpy as jnp
from jax import lax
from jax.experimental import pallas as pl
from jax.experimental.pallas import tpu as pltpu
from jax.experimental.pallas import tpu_sc as plsc

_T = 2048
_D = 1024
_E = 128
_NE = 64
_CH = 128
_TC = 64
_PAD = _TC
_TA = _T + 8 * _NE

_NC = 2
_NS = 16
_NW = _NC * _NS
_BW = _T // _NW


def _router_body(x_ref, w_ref, pos_ref, off_ref, cnt_ref):
    xv = x_ref[...]
    wv = w_ref[...]
    logits = lax.dot_general(
        xv, wv, (((1,), (1,)), ((), ())),
        preferred_element_type=jnp.float32)
    m = jnp.max(logits, axis=1, keepdims=True)
    iota_e = lax.broadcasted_iota(jnp.int32, (_T, _NE), 1)
    cand = jnp.where(logits >= m, iota_e, _NE)
    eid = jnp.min(cand, axis=1, keepdims=True)
    onehot = (iota_e == eid).astype(jnp.float32)
    counts = jnp.sum(onehot, axis=0, keepdims=True)
    offs = jnp.sum((eid < iota_e).astype(jnp.float32), axis=0, keepdims=True)
    pads = jnp.ceil(counts * 0.125) * 8.0 - counts
    ie = lax.broadcasted_iota(jnp.int32, (_NE, _NE), 0)
    je = lax.broadcasted_iota(jnp.int32, (_NE, _NE), 1)
    ult = (ie < je).astype(jnp.float32)
    pad_excl = lax.dot_general(
        pads, ult, (((1,), (0,)), ((), ())),
        preferred_element_type=jnp.float32)
    offs = offs + pad_excl
    ir = lax.broadcasted_iota(jnp.int32, (_CH, _CH), 0)
    jr = lax.broadcasted_iota(jnp.int32, (_CH, _CH), 1)
    tril = (jr < ir).astype(jnp.float32)
    running = jnp.zeros((1, _NE), jnp.float32)
    for i in range(_T // _CH):
        oh = onehot[i * _CH:(i + 1) * _CH, :]
        ranks = lax.dot_general(
            tril, oh, (((1,), (0,)), ((), ())),
            preferred_element_type=jnp.float32) + running
        posc = jnp.sum(oh * (ranks + offs), axis=1, keepdims=True)
        pos_ref[i * _CH:(i + 1) * _CH, :] = posc.astype(jnp.int32)
        running = running + jnp.sum(oh, axis=0, keepdims=True)
    off_ref[...] = offs.astype(jnp.int32)
    cnt_ref[...] = counts.astype(jnp.int32)


def _ffn_body(offs_ref, cnt_ref, scale_ref, x_hbm, ws_hbm, wl_hbm, wo_hbm,
              out_ref, xv, wsb, wlb, wob, sems, sem_x):
    def start_pair(p, slot):
        pltpu.make_async_copy(
            ws_hbm.at[pl.ds(2 * p, 2)], wsb.at[slot], sems.at[slot, 0]).start()
        pltpu.make_async_copy(
            wl_hbm.at[pl.ds(2 * p, 2)], wlb.at[slot], sems.at[slot, 1]).start()
        pltpu.make_async_copy(
            wo_hbm.at[pl.ds(2 * p, 2)], wob.at[slot], sems.at[slot, 2]).start()

    def wait_pair(p, slot):
        pltpu.make_async_copy(
            ws_hbm.at[pl.ds(2 * p, 2)], wsb.at[slot], sems.at[slot, 0]).wait()
        pltpu.make_async_copy(
            wl_hbm.at[pl.ds(2 * p, 2)], wlb.at[slot], sems.at[slot, 1]).wait()
        pltpu.make_async_copy(
            wo_hbm.at[pl.ds(2 * p, 2)], wob.at[slot], sems.at[slot, 2]).wait()

    pltpu.make_async_copy(x_hbm, xv, sem_x).start()
    for p0 in range(7):
        start_pair(p0, p0)
    pltpu.make_async_copy(x_hbm, xv, sem_x).wait()
    sc = scale_ref[0]
    npairs = _NE // 2

    def compute_expert(e, wsk, wlk, wok):
        start = pl.multiple_of(offs_ref[e], 8)
        n = cnt_ref[e]

        def chunk_body(c, carry):
            base = start + c * _TC
            xs = xv[pl.ds(base, _TC), :]
            a = jnp.dot(xs, wsk, preferred_element_type=jnp.float32)
            g = a * (1.0 / (1.0 + jnp.exp(-a)))
            u = jnp.dot(xs, wlk, preferred_element_type=jnp.float32)
            o = jnp.dot(g * u, wok, preferred_element_type=jnp.float32)
            out_ref[pl.ds(base, _TC), :] = o * sc
            return carry

        lax.fori_loop(0, (n + (_TC - 1)) // _TC, chunk_body, 0)

    def oct_body(q, carry):
        for j in range(8):
            p = 8 * q + j

            @pl.when(p + 7 < npairs)
            def _():
                start_pair(p + 7, (j + 7) % 8)

            wait_pair(p, j)
            for kk in (0, 1):
                compute_expert(2 * p + kk, wsb[j, kk], wlb[j, kk], wob[j, kk])
        return carry

    lax.fori_loop(0, npairs // 8, oct_body, 0)


@functools.lru_cache(maxsize=None)
def _make_sc_permute(n_rows_out, indexed_input):
    mesh = plsc.VectorSubcoreMesh(
        core_axis_name="c", subcore_axis_name="s", num_cores=_NC)

    @functools.partial(
        pl.kernel,
        mesh=mesh,
        out_type=jax.ShapeDtypeStruct((n_rows_out, _D), jnp.float32),
        scratch_types=[
            pltpu.VMEM((_BW,), jnp.int32),
            pltpu.VMEM((_BW, _D), jnp.float32),
            pltpu.SemaphoreType.DMA,
        ],
    )
    def permute_k(src_hbm, pos_hbm, out_hbm, idx_v, rows_v, sem):
        wid = lax.axis_index("s") * _NC + lax.axis_index("c")
        base = wid * _BW
        pltpu.sync_copy(pos_hbm.at[pl.ds(base, _BW)], idx_v)
        if indexed_input:
            pltpu.async_copy(src_hbm.at[idx_v], rows_v, sem).wait()
            pltpu.sync_copy(rows_v, out_hbm.at[pl.ds(base, _BW)])
        else:
            pltpu.sync_copy(src_hbm.at[pl.ds(base, _BW)], rows_v)
            pltpu.async_copy(rows_v, out_hbm.at[idx_v], sem).wait()

    return permute_k


def _sc_scatter(src, pos):
    return _make_sc_permute(_TA + _PAD, indexed_input=False)(src, pos)


def _sc_gather(src, pos):
    return _make_sc_permute(_T, indexed_input=True)(src, pos)


def kernel(x, router_w, expert_down, expert_up, expert_gate, scale):
    b, t, d = x.shape
    x2 = x.reshape(t, d)
    pos2d, off2d, cnt2d = pl.pallas_call(
        _router_body,
        out_shape=(
            jax.ShapeDtypeStruct((_T, 1), jnp.int32),
            jax.ShapeDtypeStruct((1, _NE), jnp.int32),
            jax.ShapeDtypeStruct((1, _NE), jnp.int32),
        ),
    )(x2, router_w)
    pos = pos2d.reshape(_T)
    x_sorted = _sc_scatter(x2, pos)
    out_sorted = pl.pallas_call(
        _ffn_body,
        in_specs=[
            pl.BlockSpec(memory_space=pltpu.SMEM),
            pl.BlockSpec(memory_space=pltpu.SMEM),
            pl.BlockSpec(memory_space=pltpu.SMEM),
            pl.BlockSpec(memory_space=pl.ANY),
            pl.BlockSpec(memory_space=pl.ANY),
            pl.BlockSpec(memory_space=pl.ANY),
            pl.BlockSpec(memory_space=pl.ANY),
        ],
        out_specs=pl.BlockSpec(memory_space=pltpu.VMEM),
        out_shape=jax.ShapeDtypeStruct((_TA + _PAD, _D), jnp.float32),
        scratch_shapes=[
            pltpu.VMEM((_TA + _PAD, _D), jnp.float32),
            pltpu.VMEM((8, 2, _D, _E), jnp.float32),
            pltpu.VMEM((8, 2, _D, _E), jnp.float32),
            pltpu.VMEM((8, 2, _E, _D), jnp.float32),
            pltpu.SemaphoreType.DMA((8, 3)),
            pltpu.SemaphoreType.DMA,
        ],
    )(off2d.reshape(_NE), cnt2d.reshape(_NE), jnp.reshape(scale, (1,)),
      x_sorted, expert_down, expert_gate, expert_up)
    out2 = _sc_gather(out_sorted, pos)
    return out2.reshape(b, t, d)


# speedup vs baseline: 1.0406x; 1.0406x over previous
import functools

import jax
import jax.numpy as jnp
from jax import lax
from jax.experimental import pallas as pl
from jax.experimental.pallas import tpu as pltpu
from jax.experimental.pallas import tpu_sc as plsc

_T = 2048
_D = 1024
_E = 128
_NE = 64
_CH = 128
_TC = 64
_PAD = _TC
_TA = _T + 8 * _NE

_NC = 2
_NS = 16
_NW = _NC * _NS
_BW = _T // _NW


def _router_body(x_ref, w_ref, pos_ref, off_ref, cnt_ref):
    xv = x_ref[...]
    wv = w_ref[...]
    logits = lax.dot_general(
        xv, wv, (((1,), (1,)), ((), ())),
        preferred_element_type=jnp.float32)
    m = jnp.max(logits, axis=1, keepdims=True)
    iota_e = lax.broadcasted_iota(jnp.int32, (_T, _NE), 1)
    cand = jnp.where(logits >= m, iota_e, _NE)
    eid = jnp.min(cand, axis=1, keepdims=True)
    onehot = (iota_e == eid).astype(jnp.float32)
    counts = jnp.sum(onehot, axis=0, keepdims=True)
    offs = jnp.sum((eid < iota_e).astype(jnp.float32), axis=0, keepdims=True)
    pads = jnp.ceil(counts * 0.125) * 8.0 - counts
    ie = lax.broadcasted_iota(jnp.int32, (_NE, _NE), 0)
    je = lax.broadcasted_iota(jnp.int32, (_NE, _NE), 1)
    ult = (ie < je).astype(jnp.float32)
    pad_excl = lax.dot_general(
        pads, ult, (((1,), (0,)), ((), ())),
        preferred_element_type=jnp.float32)
    offs = offs + pad_excl
    ir = lax.broadcasted_iota(jnp.int32, (_CH, _CH), 0)
    jr = lax.broadcasted_iota(jnp.int32, (_CH, _CH), 1)
    tril = (jr < ir).astype(jnp.float32)
    running = jnp.zeros((1, _NE), jnp.float32)
    for i in range(_T // _CH):
        oh = onehot[i * _CH:(i + 1) * _CH, :]
        ranks = lax.dot_general(
            tril, oh, (((1,), (0,)), ((), ())),
            preferred_element_type=jnp.float32) + running
        posc = jnp.sum(oh * (ranks + offs), axis=1, keepdims=True)
        pos_ref[i * _CH:(i + 1) * _CH, :] = posc.astype(jnp.int32)
        running = running + jnp.sum(oh, axis=0, keepdims=True)
    off_ref[...] = offs.astype(jnp.int32)
    cnt_ref[...] = counts.astype(jnp.int32)


def _ffn_body(offs_ref, cnt_ref, scale_ref, x_hbm, ws_hbm, wl_hbm, wo_hbm,
              out_ref, xv, wsb, wlb, wob, sems, sem_x):
    def start_pair(p, slot):
        pltpu.make_async_copy(
            ws_hbm.at[pl.ds(2 * p, 2)], wsb.at[slot], sems.at[slot, 0]).start()
        pltpu.make_async_copy(
            wl_hbm.at[pl.ds(2 * p, 2)], wlb.at[slot], sems.at[slot, 1]).start()
        pltpu.make_async_copy(
            wo_hbm.at[pl.ds(2 * p, 2)], wob.at[slot], sems.at[slot, 2]).start()

    def wait_pair(p, slot):
        pltpu.make_async_copy(
            ws_hbm.at[pl.ds(2 * p, 2)], wsb.at[slot], sems.at[slot, 0]).wait()
        pltpu.make_async_copy(
            wl_hbm.at[pl.ds(2 * p, 2)], wlb.at[slot], sems.at[slot, 1]).wait()
        pltpu.make_async_copy(
            wo_hbm.at[pl.ds(2 * p, 2)], wob.at[slot], sems.at[slot, 2]).wait()

    pltpu.make_async_copy(x_hbm, xv, sem_x).start()
    for p0 in range(3):
        start_pair(p0, p0)
    pltpu.make_async_copy(x_hbm, xv, sem_x).wait()
    sc = scale_ref[0]
    npairs = _NE // 2

    def compute_expert(e, wsk, wlk, wok):
        start = pl.multiple_of(offs_ref[e], 8)
        n = cnt_ref[e]

        def chunk_body(c, carry):
            base = start + c * _TC
            xs = xv[pl.ds(base, _TC), :].astype(jnp.bfloat16)
            a = jnp.dot(xs, wsk.astype(jnp.bfloat16),
                        preferred_element_type=jnp.float32)
            g = a * (1.0 / (1.0 + jnp.exp(-a)))
            u = jnp.dot(xs, wlk.astype(jnp.bfloat16),
                        preferred_element_type=jnp.float32)
            o = jnp.dot((g * u).astype(jnp.bfloat16),
                        wok.astype(jnp.bfloat16),
                        preferred_element_type=jnp.float32)
            out_ref[pl.ds(base, _TC), :] = o * sc
            return carry

        lax.fori_loop(0, (n + (_TC - 1)) // _TC, chunk_body, 0)

    def quad_body(q, carry):
        for j in range(4):
            p = 4 * q + j

            @pl.when(p + 3 < npairs)
            def _():
                start_pair(p + 3, (j + 3) % 4)

            wait_pair(p, j)
            for kk in (0, 1):
                compute_expert(2 * p + kk, wsb[j, kk], wlb[j, kk], wob[j, kk])
        return carry

    lax.fori_loop(0, npairs // 4, quad_body, 0)


@functools.lru_cache(maxsize=None)
def _make_sc_permute(n_rows_out, indexed_input):
    mesh = plsc.VectorSubcoreMesh(
        core_axis_name="c", subcore_axis_name="s", num_cores=_NC)

    @functools.partial(
        pl.kernel,
        mesh=mesh,
        out_type=jax.ShapeDtypeStruct((n_rows_out, _D), jnp.float32),
        scratch_types=[
            pltpu.VMEM((_BW,), jnp.int32),
            pltpu.VMEM((_BW, _D), jnp.float32),
            pltpu.SemaphoreType.DMA,
        ],
    )
    def permute_k(src_hbm, pos_hbm, out_hbm, idx_v, rows_v, sem):
        wid = lax.axis_index("s") * _NC + lax.axis_index("c")
        base = wid * _BW
        pltpu.sync_copy(pos_hbm.at[pl.ds(base, _BW)], idx_v)
        if indexed_input:
            pltpu.async_copy(src_hbm.at[idx_v], rows_v, sem).wait()
            pltpu.sync_copy(rows_v, out_hbm.at[pl.ds(base, _BW)])
        else:
            pltpu.sync_copy(src_hbm.at[pl.ds(base, _BW)], rows_v)
            pltpu.async_copy(rows_v, out_hbm.at[idx_v], sem).wait()

    return permute_k


def _sc_scatter(src, pos):
    return _make_sc_permute(_TA + _PAD, indexed_input=False)(src, pos)


def _sc_gather(src, pos):
    return _make_sc_permute(_T, indexed_input=True)(src, pos)


def kernel(x, router_w, expert_down, expert_up, expert_gate, scale):
    b, t, d = x.shape
    x2 = x.reshape(t, d)
    pos2d, off2d, cnt2d = pl.pallas_call(
        _router_body,
        out_shape=(
            jax.ShapeDtypeStruct((_T, 1), jnp.int32),
            jax.ShapeDtypeStruct((1, _NE), jnp.int32),
            jax.ShapeDtypeStruct((1, _NE), jnp.int32),
        ),
    )(x2, router_w)
    pos = pos2d.reshape(_T)
    x_sorted = _sc_scatter(x2, pos)
    out_sorted = pl.pallas_call(
        _ffn_body,
        in_specs=[
            pl.BlockSpec(memory_space=pltpu.SMEM),
            pl.BlockSpec(memory_space=pltpu.SMEM),
            pl.BlockSpec(memory_space=pltpu.SMEM),
            pl.BlockSpec(memory_space=pl.ANY),
            pl.BlockSpec(memory_space=pl.ANY),
            pl.BlockSpec(memory_space=pl.ANY),
            pl.BlockSpec(memory_space=pl.ANY),
        ],
        out_specs=pl.BlockSpec(memory_space=pltpu.VMEM),
        out_shape=jax.ShapeDtypeStruct((_TA + _PAD, _D), jnp.float32),
        scratch_shapes=[
            pltpu.VMEM((_TA + _PAD, _D), jnp.float32),
            pltpu.VMEM((4, 2, _D, _E), jnp.float32),
            pltpu.VMEM((4, 2, _D, _E), jnp.float32),
            pltpu.VMEM((4, 2, _E, _D), jnp.float32),
            pltpu.SemaphoreType.DMA((4, 3)),
            pltpu.SemaphoreType.DMA,
        ],
    )(off2d.reshape(_NE), cnt2d.reshape(_NE), jnp.reshape(scale, (1,)),
      x_sorted, expert_down, expert_gate, expert_up)
    out2 = _sc_gather(out_sorted, pos)
    return out2.reshape(b, t, d)


# speedup vs baseline: 1.0637x; 1.0221x over previous
import functools

import jax
import jax.numpy as jnp
from jax import lax
from jax.experimental import pallas as pl
from jax.experimental.pallas import tpu as pltpu
from jax.experimental.pallas import tpu_sc as plsc

_T = 2048
_D = 1024
_E = 128
_NE = 64
_CH = 128
_TC = 64
_PAD = _TC
_TA = _T + 8 * _NE

_NC = 2
_NS = 16
_NW = _NC * _NS
_BW = _T // _NW


def _router_body(x_ref, w_ref, pos_ref, off_ref, cnt_ref):
    xv = x_ref[...]
    wv = w_ref[...]
    logits = lax.dot_general(
        xv, wv, (((1,), (1,)), ((), ())),
        preferred_element_type=jnp.float32)
    m = jnp.max(logits, axis=1, keepdims=True)
    iota_e = lax.broadcasted_iota(jnp.int32, (_T, _NE), 1)
    cand = jnp.where(logits >= m, iota_e, _NE)
    eid = jnp.min(cand, axis=1, keepdims=True)
    onehot = (iota_e == eid).astype(jnp.float32)
    counts = jnp.sum(onehot, axis=0, keepdims=True)
    offs = jnp.sum((eid < iota_e).astype(jnp.float32), axis=0, keepdims=True)
    pads = jnp.ceil(counts * 0.125) * 8.0 - counts
    ie = lax.broadcasted_iota(jnp.int32, (_NE, _NE), 0)
    je = lax.broadcasted_iota(jnp.int32, (_NE, _NE), 1)
    ult = (ie < je).astype(jnp.float32)
    pad_excl = lax.dot_general(
        pads, ult, (((1,), (0,)), ((), ())),
        preferred_element_type=jnp.float32)
    offs = offs + pad_excl
    ir = lax.broadcasted_iota(jnp.int32, (_CH, _CH), 0)
    jr = lax.broadcasted_iota(jnp.int32, (_CH, _CH), 1)
    tril = (jr < ir).astype(jnp.float32)
    running = jnp.zeros((1, _NE), jnp.float32)
    for i in range(_T // _CH):
        oh = onehot[i * _CH:(i + 1) * _CH, :]
        ranks = lax.dot_general(
            tril, oh, (((1,), (0,)), ((), ())),
            preferred_element_type=jnp.float32) + running
        posc = jnp.sum(oh * (ranks + offs), axis=1, keepdims=True)
        pos_ref[i * _CH:(i + 1) * _CH, :] = posc.astype(jnp.int32)
        running = running + jnp.sum(oh, axis=0, keepdims=True)
    off_ref[...] = offs.astype(jnp.int32)
    cnt_ref[...] = counts.astype(jnp.int32)


def _ffn_body(offs_ref, cnt_ref, scale_ref, x_hbm, ws_hbm, wl_hbm, wo_hbm,
              out_ref, xv, wsb, wlb, wob, obuf, sems, osems, sem_x):
    def start_pair(p, slot):
        pltpu.make_async_copy(
            ws_hbm.at[pl.ds(2 * p, 2)], wsb.at[slot], sems.at[slot, 0]).start()
        pltpu.make_async_copy(
            wl_hbm.at[pl.ds(2 * p, 2)], wlb.at[slot], sems.at[slot, 1]).start()
        pltpu.make_async_copy(
            wo_hbm.at[pl.ds(2 * p, 2)], wob.at[slot], sems.at[slot, 2]).start()

    def wait_pair(p, slot):
        pltpu.make_async_copy(
            ws_hbm.at[pl.ds(2 * p, 2)], wsb.at[slot], sems.at[slot, 0]).wait()
        pltpu.make_async_copy(
            wl_hbm.at[pl.ds(2 * p, 2)], wlb.at[slot], sems.at[slot, 1]).wait()
        pltpu.make_async_copy(
            wo_hbm.at[pl.ds(2 * p, 2)], wob.at[slot], sems.at[slot, 2]).wait()

    pltpu.make_async_copy(x_hbm, xv, sem_x).start()
    for p0 in range(3):
        start_pair(p0, p0)
    pltpu.make_async_copy(x_hbm, xv, sem_x).wait()
    sc = scale_ref[0]
    npairs = _NE // 2

    def compute_expert(e, wsk, wlk, wok, cglob):
        start = pl.multiple_of(offs_ref[e], 8)
        n = cnt_ref[e]

        def chunk_body(c, cg):
            base = start + c * _TC
            slot = lax.rem(cg, 4)
            xs = xv[pl.ds(base, _TC), :].astype(jnp.bfloat16)
            a = jnp.dot(xs, wsk.astype(jnp.bfloat16),
                        preferred_element_type=jnp.float32)
            g = a * (1.0 / (1.0 + jnp.exp(-a)))
            u = jnp.dot(xs, wlk.astype(jnp.bfloat16),
                        preferred_element_type=jnp.float32)
            o = jnp.dot((g * u).astype(jnp.bfloat16),
                        wok.astype(jnp.bfloat16),
                        preferred_element_type=jnp.float32)

            @pl.when(cg >= 4)
            def _():
                pltpu.make_async_copy(
                    obuf.at[slot], out_ref.at[pl.ds(base, _TC)],
                    osems.at[slot]).wait()

            obuf[slot] = o * sc
            pltpu.make_async_copy(
                obuf.at[slot], out_ref.at[pl.ds(base, _TC)],
                osems.at[slot]).start()
            return cg + 1

        return lax.fori_loop(0, (n + (_TC - 1)) // _TC, chunk_body, cglob)

    def quad_body(q, cglob):
        for j in range(4):
            p = 4 * q + j

            @pl.when(p + 3 < npairs)
            def _():
                start_pair(p + 3, (j + 3) % 4)

            wait_pair(p, j)
            for kk in (0, 1):
                cglob = compute_expert(
                    2 * p + kk, wsb[j, kk], wlb[j, kk], wob[j, kk], cglob)
        return cglob

    nch_total = lax.fori_loop(0, npairs // 4, quad_body, 0)
    for s_ in range(4):
        @pl.when(jnp.logical_or(nch_total >= 4, s_ < nch_total))
        def _():
            pltpu.make_async_copy(
                obuf.at[s_], out_ref.at[pl.ds(0, _TC)], osems.at[s_]).wait()


@functools.lru_cache(maxsize=None)
def _make_sc_permute(n_rows_out, indexed_input):
    mesh = plsc.VectorSubcoreMesh(
        core_axis_name="c", subcore_axis_name="s", num_cores=_NC)

    @functools.partial(
        pl.kernel,
        mesh=mesh,
        out_type=jax.ShapeDtypeStruct((n_rows_out, _D), jnp.float32),
        scratch_types=[
            pltpu.VMEM((_BW,), jnp.int32),
            pltpu.VMEM((_BW, _D), jnp.float32),
            pltpu.SemaphoreType.DMA,
        ],
    )
    def permute_k(src_hbm, pos_hbm, out_hbm, idx_v, rows_v, sem):
        wid = lax.axis_index("s") * _NC + lax.axis_index("c")
        base = wid * _BW
        pltpu.sync_copy(pos_hbm.at[pl.ds(base, _BW)], idx_v)
        if indexed_input:
            pltpu.async_copy(src_hbm.at[idx_v], rows_v, sem).wait()
            pltpu.sync_copy(rows_v, out_hbm.at[pl.ds(base, _BW)])
        else:
            pltpu.sync_copy(src_hbm.at[pl.ds(base, _BW)], rows_v)
            pltpu.async_copy(rows_v, out_hbm.at[idx_v], sem).wait()

    return permute_k


def _sc_scatter(src, pos):
    return _make_sc_permute(_TA + _PAD, indexed_input=False)(src, pos)


def _sc_gather(src, pos):
    return _make_sc_permute(_T, indexed_input=True)(src, pos)


def kernel(x, router_w, expert_down, expert_up, expert_gate, scale):
    b, t, d = x.shape
    x2 = x.reshape(t, d)
    pos2d, off2d, cnt2d = pl.pallas_call(
        _router_body,
        out_shape=(
            jax.ShapeDtypeStruct((_T, 1), jnp.int32),
            jax.ShapeDtypeStruct((1, _NE), jnp.int32),
            jax.ShapeDtypeStruct((1, _NE), jnp.int32),
        ),
    )(x2, router_w)
    pos = pos2d.reshape(_T)
    x_sorted = _sc_scatter(x2, pos)
    out_sorted = pl.pallas_call(
        _ffn_body,
        in_specs=[
            pl.BlockSpec(memory_space=pltpu.SMEM),
            pl.BlockSpec(memory_space=pltpu.SMEM),
            pl.BlockSpec(memory_space=pltpu.SMEM),
            pl.BlockSpec(memory_space=pl.ANY),
            pl.BlockSpec(memory_space=pl.ANY),
            pl.BlockSpec(memory_space=pl.ANY),
            pl.BlockSpec(memory_space=pl.ANY),
        ],
        out_specs=pl.BlockSpec(memory_space=pl.ANY),
        out_shape=jax.ShapeDtypeStruct((_TA + _PAD, _D), jnp.float32),
        scratch_shapes=[
            pltpu.VMEM((_TA + _PAD, _D), jnp.float32),
            pltpu.VMEM((4, 2, _D, _E), jnp.float32),
            pltpu.VMEM((4, 2, _D, _E), jnp.float32),
            pltpu.VMEM((4, 2, _E, _D), jnp.float32),
            pltpu.VMEM((4, _TC, _D), jnp.float32),
            pltpu.SemaphoreType.DMA((4, 3)),
            pltpu.SemaphoreType.DMA((4,)),
            pltpu.SemaphoreType.DMA,
        ],
    )(off2d.reshape(_NE), cnt2d.reshape(_NE), jnp.reshape(scale, (1,)),
      x_sorted, expert_down, expert_gate, expert_up)
    out2 = _sc_gather(out_sorted, pos)
    return out2.reshape(b, t, d)
